# baseline (speedup 1.0000x reference)
import functools

import jax
import jax.numpy as jnp
from jax import lax
from jax.experimental import pallas as pl
from jax.experimental.pallas import tpu as pltpu
from jax.experimental.pallas import tpu_sc as plsc

_N = 10000
_E = 320000
_D = 128
_H = 128
_G = 64
_AUGW = 144
_CHUNK = 128
_NCHUNKS = _E // _CHUNK
_NW = 32
_NPAD = 10240
_ROWS_PER_TILE = _NPAD // 16
_BLK = 1000
_NBLK = _N // _BLK


_EPAD = 327680
_NCH_W = 80


def _sc_edge_agg_body(yl_hbm, src_hbm, dst_hbm, zeros_hbm, out_hbm,
                      src_a, dst_a, src_b, dst_b, src_a2, dst_a2,
                      buf_a, buf_b, agg_sh, sem_a, sem_b, sem_i):
    c = lax.axis_index("c")
    s = lax.axis_index("s")
    wid = s * 2 + c
    pltpu.sync_copy(zeros_hbm,
                    agg_sh.at[pl.ds(s * _ROWS_PER_TILE, _ROWS_PER_TILE)])
    plsc.subcore_barrier()

    ebase = wid * (_NCH_W * _CHUNK)

    def load_idx(j, sref, dref):
        pltpu.sync_copy(src_hbm.at[pl.ds(ebase + j * _CHUNK, _CHUNK)], sref)
        pltpu.sync_copy(dst_hbm.at[pl.ds(ebase + j * _CHUNK, _CHUNK)], dref)

    def start_gather(sref, buf, sem):
        pltpu.async_copy(yl_hbm.at[sref], buf, sem)

    def wait_gather(sref, buf, sem):
        pltpu.make_async_copy(yl_hbm.at[sref], buf, sem).wait()

    def body(t, carry):
        load_idx(t, src_a, dst_a)
        pltpu.async_copy(yl_hbm.at[src_a], buf_a, sem_a).wait()
        pltpu.sync_copy(buf_a, agg_sh.at[dst_a], add=True)
        return carry

    lax.fori_loop(0, _NCH_W, body, 0)
    plsc.subcore_barrier()
    pltpu.sync_copy(agg_sh.at[pl.ds(s * _ROWS_PER_TILE, _ROWS_PER_TILE)],
                    out_hbm.at[c, pl.ds(s * _ROWS_PER_TILE, _ROWS_PER_TILE)])


def _edge_agg(yl_aug, src2d, dst2d, zeros):
    call = pl.kernel(
        _sc_edge_agg_body,
        out_type=jax.ShapeDtypeStruct((2, _NPAD, _AUGW), jnp.float32),
        mesh=plsc.VectorSubcoreMesh(core_axis_name="c", subcore_axis_name="s"),
        scratch_types=[
            pltpu.VMEM((_CHUNK,), jnp.int32),
            pltpu.VMEM((_CHUNK,), jnp.int32),
            pltpu.VMEM((_CHUNK,), jnp.int32),
            pltpu.VMEM((_CHUNK,), jnp.int32),
            pltpu.VMEM((_CHUNK,), jnp.int32),
            pltpu.VMEM((_CHUNK,), jnp.int32),
            pltpu.VMEM((_CHUNK, _AUGW), jnp.float32),
            pltpu.VMEM((_CHUNK, _AUGW), jnp.float32),
            pltpu.VMEM_SHARED((_NPAD, _AUGW), jnp.float32),
            pltpu.SemaphoreType.DMA,
            pltpu.SemaphoreType.DMA,
            pltpu.SemaphoreType.DMA,
        ],
        compiler_params=pltpu.CompilerParams(use_tc_tiling_on_sc=False),
    )
    return call(yl_aug, src2d, dst2d, zeros)


def _k1_body(nb, eb, wla, wr, w1, w2, bcol, blr, b1r, b2r,
             yl_out, base_out, mlp_out):
    xb = jnp.concatenate([nb[...], eb[...]], axis=1)
    yl_out[...] = jnp.dot(xb, wla[...], preferred_element_type=jnp.float32) + bcol[...]
    base_out[...] = jnp.dot(xb, wr[...], preferred_element_type=jnp.float32) + blr[...]
    h1 = jnp.maximum(jnp.dot(xb, w1[...], preferred_element_type=jnp.float32) + b1r[...], 0.0)
    mlp_out[...] = jnp.maximum(jnp.dot(h1, w2[...], preferred_element_type=jnp.float32) + b2r[...], 0.0)


def _k1(nf, ef, wla, wr, w1, w2, bcol, blr, b1r, b2r):
    return pl.pallas_call(
        _k1_body,
        grid=(_NBLK,),
        in_specs=[
            pl.BlockSpec((_BLK, _D), lambda i: (i, 0)),
            pl.BlockSpec((_BLK, _D), lambda i: (i, 0)),
            pl.BlockSpec((2 * _D, _AUGW), lambda i: (0, 0)),
            pl.BlockSpec((2 * _D, _H), lambda i: (0, 0)),
            pl.BlockSpec((2 * _D, _H), lambda i: (0, 0)),
            pl.BlockSpec((_H, _H), lambda i: (0, 0)),
            pl.BlockSpec((1, _AUGW), lambda i: (0, 0)),
            pl.BlockSpec((1, _H), lambda i: (0, 0)),
            pl.BlockSpec((1, _H), lambda i: (0, 0)),
            pl.BlockSpec((1, _H), lambda i: (0, 0)),
        ],
        out_specs=[
            pl.BlockSpec((_BLK, _AUGW), lambda i: (i, 0)),
            pl.BlockSpec((_BLK, _H), lambda i: (i, 0)),
            pl.BlockSpec((_BLK, _H), lambda i: (i, 0)),
        ],
        out_shape=[
            jax.ShapeDtypeStruct((_N, _AUGW), jnp.float32),
            jax.ShapeDtypeStruct((_N, _H), jnp.float32),
            jax.ShapeDtypeStruct((_N, _H), jnp.float32),
        ],
    )(nf, ef, wla, wr, w1, w2, bcol, blr, b1r, b2r)


def _k3_body(a0, a1, baseb, pre_out, sums, sumsq):
    i = pl.program_id(0)
    aggb = a0[...] + a1[...]
    deg = jnp.maximum(aggb[:, _H:_H + 1], 1.0)
    pre = aggb[:, :_H] / deg + baseb[...]
    pre_out[...] = pre

    @pl.when(i == 0)
    def _():
        sums[...] = jnp.zeros_like(sums)
        sumsq[...] = jnp.zeros_like(sumsq)

    sums[...] += jnp.sum(pre, axis=0, keepdims=True)
    sumsq[...] += jnp.sum(pre * pre, axis=0, keepdims=True)


def _k3(a0, a1, base):
    return pl.pallas_call(
        _k3_body,
        grid=(_NBLK,),
        in_specs=[
            pl.BlockSpec((_BLK, _AUGW), lambda i: (i, 0)),
            pl.BlockSpec((_BLK, _AUGW), lambda i: (i, 0)),
            pl.BlockSpec((_BLK, _H), lambda i: (i, 0)),
        ],
        out_specs=[
            pl.BlockSpec((_BLK, _H), lambda i: (i, 0)),
            pl.BlockSpec((1, _H), lambda i: (0, 0)),
            pl.BlockSpec((1, _H), lambda i: (0, 0)),
        ],
        out_shape=[
            jax.ShapeDtypeStruct((_N, _H), jnp.float32),
            jax.ShapeDtypeStruct((1, _H), jnp.float32),
            jax.ShapeDtypeStruct((1, _H), jnp.float32),
        ],
    )(a0, a1, base)


def _k4_body(preb, mlpb, batchb, sums, sumsq, gam, bet, wf, bfr,
             out, gacc, cacc):
    i = pl.program_id(0)
    mu = sums[...] / _N
    var = sumsq[...] / _N - mu * mu
    rstd = lax.rsqrt(var + 1e-5)
    xg = (preb[...] - mu) * rstd * gam[...] + bet[...]
    comb = jnp.maximum(xg, 0.0) + mlpb[...]
    b = batchb[0]
    gi = lax.broadcasted_iota(jnp.int32, (_G, 1), 0)
    oh = (gi == b).astype(jnp.float32)

    @pl.when(i == 0)
    def _():
        gacc[...] = jnp.zeros_like(gacc)
        cacc[...] = jnp.zeros_like(cacc)

    gacc[...] += jnp.dot(oh, comb, preferred_element_type=jnp.float32)
    cacc[...] += jnp.sum(oh, axis=1, keepdims=True)

    @pl.when(i == pl.num_programs(0) - 1)
    def _():
        gf = gacc[...] / jnp.maximum(cacc[...], 1.0)
        z = jnp.dot(gf, wf[...], preferred_element_type=jnp.float32) + bfr[...]
        out[...] = jax.nn.sigmoid(z)


def _k4(pre, mlp, batch3, sums, sumsq, gam, bet, wf, bfr):
    return pl.pallas_call(
        _k4_body,
        grid=(_NBLK,),
        in_specs=[
            pl.BlockSpec((_BLK, _H), lambda i: (i, 0)),
            pl.BlockSpec((_BLK, _H), lambda i: (i, 0)),
            pl.BlockSpec((1, 1, _BLK), lambda i: (i, 0, 0)),
            pl.BlockSpec((1, _H), lambda i: (0, 0)),
            pl.BlockSpec((1, _H), lambda i: (0, 0)),
            pl.BlockSpec((1, _H), lambda i: (0, 0)),
            pl.BlockSpec((1, _H), lambda i: (0, 0)),
            pl.BlockSpec((_H, 1), lambda i: (0, 0)),
            pl.BlockSpec((1, 1), lambda i: (0, 0)),
        ],
        out_specs=pl.BlockSpec((_G, 1), lambda i: (0, 0)),
        out_shape=jax.ShapeDtypeStruct((_G, 1), jnp.float32),
        scratch_shapes=[
            pltpu.VMEM((_G, _H), jnp.float32),
            pltpu.VMEM((_G, 1), jnp.float32),
        ],
    )(pre, mlp, batch3, sums, sumsq, gam, bet, wf, bfr)


def kernel(normal_features, extreme_features, Wl, bl, Wr, gamma, beta,
           W1, b1, W2, b2, Wf, bf, edge_index, batch):
    f32 = jnp.float32
    wla = jnp.concatenate([Wl, jnp.zeros((2 * _D, _AUGW - _H), f32)], axis=1)
    bcol = jnp.zeros((1, _AUGW), f32).at[0, _H].set(1.0)
    blr = bl.reshape(1, _H)
    b1r = b1.reshape(1, _H)
    b2r = b2.reshape(1, _H)
    bfr = bf.reshape(1, 1)
    gam = gamma.reshape(1, _H)
    bet = beta.reshape(1, _H)

    yl_aug, base, mlp = _k1(normal_features, extreme_features,
                            wla, Wr, W1, W2, bcol, blr, b1r, b2r)

    zeros = jnp.zeros((_ROWS_PER_TILE, _AUGW), f32)
    npad_e = _EPAD - _E
    src1 = jnp.concatenate([edge_index[0], jnp.zeros((npad_e,), jnp.int32)])
    dst1 = jnp.concatenate(
        [edge_index[1], jnp.full((npad_e,), _NPAD - 1, jnp.int32)])
    agg2 = _edge_agg(yl_aug, src1, dst1, zeros)

    pre, sums, sumsq = _k3(agg2[0, :_N], agg2[1, :_N], base)

    batch3 = batch.reshape(_NBLK, 1, _BLK)
    return _k4(pre, mlp, batch3, sums, sumsq, gam, bet, Wf, bfr)


# speedup vs baseline: 2.3761x; 2.3761x over previous
import functools

import jax
import jax.numpy as jnp
from jax import lax
from jax.experimental import pallas as pl
from jax.experimental.pallas import tpu as pltpu
from jax.experimental.pallas import tpu_sc as plsc

_N = 10000
_E = 320000
_D = 128
_H = 128
_G = 64
_AUGW = 144
_CHUNK = 128
_NCHUNKS = _E // _CHUNK
_NW = 32
_NPAD = 10240
_ROWS_PER_TILE = _NPAD // 16
_BLK = 1000
_NBLK = _N // _BLK


_EPAD = 327680
_NCH_W = 80


def _sc_edge_agg_body(yl_hbm, src_hbm, dst_hbm, zeros_hbm, out_hbm,
                      src_a, dst_a, src_b, dst_b, src_a2, dst_a2,
                      buf_a, buf_b, agg_sh, sem_a, sem_b, sem_i):
    c = lax.axis_index("c")
    s = lax.axis_index("s")
    wid = s * 2 + c
    pltpu.sync_copy(zeros_hbm,
                    agg_sh.at[pl.ds(s * _ROWS_PER_TILE, _ROWS_PER_TILE)])
    plsc.subcore_barrier()

    ebase = wid * (_NCH_W * _CHUNK)

    def load_idx(j, sref, dref):
        pltpu.sync_copy(src_hbm.at[pl.ds(ebase + j * _CHUNK, _CHUNK)], sref)
        pltpu.sync_copy(dst_hbm.at[pl.ds(ebase + j * _CHUNK, _CHUNK)], dref)

    def start_gather(sref, buf, sem):
        pltpu.async_copy(yl_hbm.at[sref], buf, sem)

    def wait_gather(sref, buf, sem):
        pltpu.make_async_copy(yl_hbm.at[sref], buf, sem).wait()

    load_idx(0, src_a, dst_a)
    start_gather(src_a, buf_a, sem_a)

    def body(t, carry):
        j1 = 2 * t + 1
        load_idx(j1, src_b, dst_b)
        start_gather(src_b, buf_b, sem_b)
        wait_gather(src_a, buf_a, sem_a)
        pltpu.sync_copy(buf_a, agg_sh.at[dst_a], add=True)
        load_idx(j1 + 1, src_a, dst_a)
        start_gather(src_a, buf_a, sem_a)
        wait_gather(src_b, buf_b, sem_b)
        pltpu.sync_copy(buf_b, agg_sh.at[dst_b], add=True)
        return carry

    lax.fori_loop(0, _NCH_W // 2 - 1, body, 0)
    load_idx(_NCH_W - 1, src_b, dst_b)
    start_gather(src_b, buf_b, sem_b)
    wait_gather(src_a, buf_a, sem_a)
    pltpu.sync_copy(buf_a, agg_sh.at[dst_a], add=True)
    wait_gather(src_b, buf_b, sem_b)
    pltpu.sync_copy(buf_b, agg_sh.at[dst_b], add=True)
    plsc.subcore_barrier()
    pltpu.sync_copy(agg_sh.at[pl.ds(s * _ROWS_PER_TILE, _ROWS_PER_TILE)],
                    out_hbm.at[c, pl.ds(s * _ROWS_PER_TILE, _ROWS_PER_TILE)])


def _edge_agg(yl_aug, src2d, dst2d, zeros):
    call = pl.kernel(
        _sc_edge_agg_body,
        out_type=jax.ShapeDtypeStruct((2, _NPAD, _AUGW), jnp.float32),
        mesh=plsc.VectorSubcoreMesh(core_axis_name="c", subcore_axis_name="s"),
        scratch_types=[
            pltpu.VMEM((_CHUNK,), jnp.int32),
            pltpu.VMEM((_CHUNK,), jnp.int32),
            pltpu.VMEM((_CHUNK,), jnp.int32),
            pltpu.VMEM((_CHUNK,), jnp.int32),
            pltpu.VMEM((_CHUNK,), jnp.int32),
            pltpu.VMEM((_CHUNK,), jnp.int32),
            pltpu.VMEM((_CHUNK, _AUGW), jnp.float32),
            pltpu.VMEM((_CHUNK, _AUGW), jnp.float32),
            pltpu.VMEM_SHARED((_NPAD, _AUGW), jnp.float32),
            pltpu.SemaphoreType.DMA,
            pltpu.SemaphoreType.DMA,
            pltpu.SemaphoreType.DMA,
        ],
        compiler_params=pltpu.CompilerParams(use_tc_tiling_on_sc=False),
    )
    return call(yl_aug, src2d, dst2d, zeros)


def _k1_body(nb, eb, wla, wr, w1, w2, bcol, blr, b1r, b2r,
             yl_out, base_out, mlp_out):
    xb = jnp.concatenate([nb[...], eb[...]], axis=1)
    yl_out[...] = jnp.dot(xb, wla[...], preferred_element_type=jnp.float32) + bcol[...]
    base_out[...] = jnp.dot(xb, wr[...], preferred_element_type=jnp.float32) + blr[...]
    h1 = jnp.maximum(jnp.dot(xb, w1[...], preferred_element_type=jnp.float32) + b1r[...], 0.0)
    mlp_out[...] = jnp.maximum(jnp.dot(h1, w2[...], preferred_element_type=jnp.float32) + b2r[...], 0.0)


def _k1(nf, ef, wla, wr, w1, w2, bcol, blr, b1r, b2r):
    return pl.pallas_call(
        _k1_body,
        grid=(_NBLK,),
        in_specs=[
            pl.BlockSpec((_BLK, _D), lambda i: (i, 0)),
            pl.BlockSpec((_BLK, _D), lambda i: (i, 0)),
            pl.BlockSpec((2 * _D, _AUGW), lambda i: (0, 0)),
            pl.BlockSpec((2 * _D, _H), lambda i: (0, 0)),
            pl.BlockSpec((2 * _D, _H), lambda i: (0, 0)),
            pl.BlockSpec((_H, _H), lambda i: (0, 0)),
            pl.BlockSpec((1, _AUGW), lambda i: (0, 0)),
            pl.BlockSpec((1, _H), lambda i: (0, 0)),
            pl.BlockSpec((1, _H), lambda i: (0, 0)),
            pl.BlockSpec((1, _H), lambda i: (0, 0)),
        ],
        out_specs=[
            pl.BlockSpec((_BLK, _AUGW), lambda i: (i, 0)),
            pl.BlockSpec((_BLK, _H), lambda i: (i, 0)),
            pl.BlockSpec((_BLK, _H), lambda i: (i, 0)),
        ],
        out_shape=[
            jax.ShapeDtypeStruct((_N, _AUGW), jnp.float32),
            jax.ShapeDtypeStruct((_N, _H), jnp.float32),
            jax.ShapeDtypeStruct((_N, _H), jnp.float32),
        ],
    )(nf, ef, wla, wr, w1, w2, bcol, blr, b1r, b2r)


def _k3_body(a0, a1, baseb, pre_out, sums, sumsq):
    i = pl.program_id(0)
    aggb = a0[...] + a1[...]
    deg = jnp.maximum(aggb[:, _H:_H + 1], 1.0)
    pre = aggb[:, :_H] / deg + baseb[...]
    pre_out[...] = pre

    @pl.when(i == 0)
    def _():
        sums[...] = jnp.zeros_like(sums)
        sumsq[...] = jnp.zeros_like(sumsq)

    sums[...] += jnp.sum(pre, axis=0, keepdims=True)
    sumsq[...] += jnp.sum(pre * pre, axis=0, keepdims=True)


def _k3(a0, a1, base):
    return pl.pallas_call(
        _k3_body,
        grid=(_NBLK,),
        in_specs=[
            pl.BlockSpec((_BLK, _AUGW), lambda i: (i, 0)),
            pl.BlockSpec((_BLK, _AUGW), lambda i: (i, 0)),
            pl.BlockSpec((_BLK, _H), lambda i: (i, 0)),
        ],
        out_specs=[
            pl.BlockSpec((_BLK, _H), lambda i: (i, 0)),
            pl.BlockSpec((1, _H), lambda i: (0, 0)),
            pl.BlockSpec((1, _H), lambda i: (0, 0)),
        ],
        out_shape=[
            jax.ShapeDtypeStruct((_N, _H), jnp.float32),
            jax.ShapeDtypeStruct((1, _H), jnp.float32),
            jax.ShapeDtypeStruct((1, _H), jnp.float32),
        ],
    )(a0, a1, base)


def _k4_body(preb, mlpb, batchb, sums, sumsq, gam, bet, wf, bfr,
             out, gacc, cacc):
    i = pl.program_id(0)
    mu = sums[...] / _N
    var = sumsq[...] / _N - mu * mu
    rstd = lax.rsqrt(var + 1e-5)
    xg = (preb[...] - mu) * rstd * gam[...] + bet[...]
    comb = jnp.maximum(xg, 0.0) + mlpb[...]
    b = batchb[0]
    gi = lax.broadcasted_iota(jnp.int32, (_G, 1), 0)
    oh = (gi == b).astype(jnp.float32)

    @pl.when(i == 0)
    def _():
        gacc[...] = jnp.zeros_like(gacc)
        cacc[...] = jnp.zeros_like(cacc)

    gacc[...] += jnp.dot(oh, comb, preferred_element_type=jnp.float32)
    cacc[...] += jnp.sum(oh, axis=1, keepdims=True)

    @pl.when(i == pl.num_programs(0) - 1)
    def _():
        gf = gacc[...] / jnp.maximum(cacc[...], 1.0)
        z = jnp.dot(gf, wf[...], preferred_element_type=jnp.float32) + bfr[...]
        out[...] = jax.nn.sigmoid(z)


def _k4(pre, mlp, batch3, sums, sumsq, gam, bet, wf, bfr):
    return pl.pallas_call(
        _k4_body,
        grid=(_NBLK,),
        in_specs=[
            pl.BlockSpec((_BLK, _H), lambda i: (i, 0)),
            pl.BlockSpec((_BLK, _H), lambda i: (i, 0)),
            pl.BlockSpec((1, 1, _BLK), lambda i: (i, 0, 0)),
            pl.BlockSpec((1, _H), lambda i: (0, 0)),
            pl.BlockSpec((1, _H), lambda i: (0, 0)),
            pl.BlockSpec((1, _H), lambda i: (0, 0)),
            pl.BlockSpec((1, _H), lambda i: (0, 0)),
            pl.BlockSpec((_H, 1), lambda i: (0, 0)),
            pl.BlockSpec((1, 1), lambda i: (0, 0)),
        ],
        out_specs=pl.BlockSpec((_G, 1), lambda i: (0, 0)),
        out_shape=jax.ShapeDtypeStruct((_G, 1), jnp.float32),
        scratch_shapes=[
            pltpu.VMEM((_G, _H), jnp.float32),
            pltpu.VMEM((_G, 1), jnp.float32),
        ],
    )(pre, mlp, batch3, sums, sumsq, gam, bet, wf, bfr)


def kernel(normal_features, extreme_features, Wl, bl, Wr, gamma, beta,
           W1, b1, W2, b2, Wf, bf, edge_index, batch):
    f32 = jnp.float32
    wla = jnp.concatenate([Wl, jnp.zeros((2 * _D, _AUGW - _H), f32)], axis=1)
    bcol = jnp.zeros((1, _AUGW), f32).at[0, _H].set(1.0)
    blr = bl.reshape(1, _H)
    b1r = b1.reshape(1, _H)
    b2r = b2.reshape(1, _H)
    bfr = bf.reshape(1, 1)
    gam = gamma.reshape(1, _H)
    bet = beta.reshape(1, _H)

    yl_aug, base, mlp = _k1(normal_features, extreme_features,
                            wla, Wr, W1, W2, bcol, blr, b1r, b2r)

    zeros = jnp.zeros((_ROWS_PER_TILE, _AUGW), f32)
    npad_e = _EPAD - _E
    pad_i = jnp.arange(npad_e, dtype=jnp.int32)
    src1 = jnp.concatenate([edge_index[0], pad_i % _N])
    dst1 = jnp.concatenate([edge_index[1], _N + (pad_i % (_NPAD - _N))])
    agg2 = _edge_agg(yl_aug, src1, dst1, zeros)

    pre, sums, sumsq = _k3(agg2[0, :_N], agg2[1, :_N], base)

    batch3 = batch.reshape(_NBLK, 1, _BLK)
    return _k4(pre, mlp, batch3, sums, sumsq, gam, bet, Wf, bfr)


# speedup vs baseline: 2.5618x; 1.0782x over previous
import functools

import jax
import jax.numpy as jnp
from jax import lax
from jax.experimental import pallas as pl
from jax.experimental.pallas import tpu as pltpu
from jax.experimental.pallas import tpu_sc as plsc

_N = 10000
_E = 320000
_D = 128
_H = 128
_G = 64
_AUGW = 144
_CHUNK = 128
_NCHUNKS = _E // _CHUNK
_NW = 32
_NPAD = 10240
_ROWS_PER_TILE = _NPAD // 16
_BLK = 1000
_NBLK = _N // _BLK


_NCH_W = 78


def _sc_edge_agg_body(yl_hbm, src_hbm, dst_hbm, zeros_hbm, out_hbm,
                      src_a, dst_a, src_b, dst_b,
                      buf_a, buf_b, agg_sh, sem_a, sem_b):
    c = lax.axis_index("c")
    s = lax.axis_index("s")
    wid = s * 2 + c
    pltpu.sync_copy(zeros_hbm,
                    agg_sh.at[pl.ds(s * _ROWS_PER_TILE, _ROWS_PER_TILE)])
    plsc.subcore_barrier()

    ebase = wid * (_NCH_W * _CHUNK)

    def load_idx_abs(base, sref, dref):
        pltpu.sync_copy(src_hbm.at[pl.ds(base, _CHUNK)], sref)
        pltpu.sync_copy(dst_hbm.at[pl.ds(base, _CHUNK)], dref)

    def load_idx(j, sref, dref):
        load_idx_abs(ebase + j * _CHUNK, sref, dref)

    def start_gather(sref, buf, sem):
        pltpu.async_copy(yl_hbm.at[sref], buf, sem)

    def wait_gather(sref, buf, sem):
        pltpu.make_async_copy(yl_hbm.at[sref], buf, sem).wait()

    load_idx(0, src_a, dst_a)
    start_gather(src_a, buf_a, sem_a)

    def body(t, carry):
        j1 = 2 * t + 1
        load_idx(j1, src_b, dst_b)
        start_gather(src_b, buf_b, sem_b)
        wait_gather(src_a, buf_a, sem_a)
        pltpu.sync_copy(buf_a, agg_sh.at[dst_a], add=True)
        load_idx(j1 + 1, src_a, dst_a)
        start_gather(src_a, buf_a, sem_a)
        wait_gather(src_b, buf_b, sem_b)
        pltpu.sync_copy(buf_b, agg_sh.at[dst_b], add=True)
        return carry

    lax.fori_loop(0, _NCH_W // 2 - 1, body, 0)
    load_idx(_NCH_W - 1, src_b, dst_b)
    start_gather(src_b, buf_b, sem_b)
    wait_gather(src_a, buf_a, sem_a)
    pltpu.sync_copy(buf_a, agg_sh.at[dst_a], add=True)
    wait_gather(src_b, buf_b, sem_b)
    pltpu.sync_copy(buf_b, agg_sh.at[dst_b], add=True)

    @pl.when(wid < 4)
    def _():
        load_idx_abs((_NW * _NCH_W + wid) * _CHUNK, src_a, dst_a)
        pltpu.async_copy(yl_hbm.at[src_a], buf_a, sem_a).wait()
        pltpu.sync_copy(buf_a, agg_sh.at[dst_a], add=True)

    plsc.subcore_barrier()
    pltpu.sync_copy(agg_sh.at[pl.ds(s * _ROWS_PER_TILE, _ROWS_PER_TILE)],
                    out_hbm.at[c, pl.ds(s * _ROWS_PER_TILE, _ROWS_PER_TILE)])


def _edge_agg(yl_aug, src1, dst1, zeros):
    call = pl.kernel(
        _sc_edge_agg_body,
        out_type=jax.ShapeDtypeStruct((2, _NPAD, _AUGW), jnp.float32),
        mesh=plsc.VectorSubcoreMesh(core_axis_name="c", subcore_axis_name="s"),
        scratch_types=[
            pltpu.VMEM((_CHUNK,), jnp.int32),
            pltpu.VMEM((_CHUNK,), jnp.int32),
            pltpu.VMEM((_CHUNK,), jnp.int32),
            pltpu.VMEM((_CHUNK,), jnp.int32),
            pltpu.VMEM((_CHUNK, _AUGW), jnp.float32),
            pltpu.VMEM((_CHUNK, _AUGW), jnp.float32),
            pltpu.VMEM_SHARED((_NPAD, _AUGW), jnp.float32),
            pltpu.SemaphoreType.DMA,
            pltpu.SemaphoreType.DMA,
        ],
        compiler_params=pltpu.CompilerParams(use_tc_tiling_on_sc=False),
    )
    return call(yl_aug, src1, dst1, zeros)


def _k1_body(nb, eb, wla, wr, w1, w2, bcol, blr, b1r, b2r,
             yl_out, base_out, mlp_out):
    xb = jnp.concatenate([nb[...], eb[...]], axis=1)
    yl_out[...] = jnp.dot(xb, wla[...], preferred_element_type=jnp.float32) + bcol[...]
    base_out[...] = jnp.dot(xb, wr[...], preferred_element_type=jnp.float32) + blr[...]
    h1 = jnp.maximum(jnp.dot(xb, w1[...], preferred_element_type=jnp.float32) + b1r[...], 0.0)
    mlp_out[...] = jnp.maximum(jnp.dot(h1, w2[...], preferred_element_type=jnp.float32) + b2r[...], 0.0)


def _k1(nf, ef, wla, wr, w1, w2, bcol, blr, b1r, b2r):
    return pl.pallas_call(
        _k1_body,
        grid=(_NBLK,),
        in_specs=[
            pl.BlockSpec((_BLK, _D), lambda i: (i, 0)),
            pl.BlockSpec((_BLK, _D), lambda i: (i, 0)),
            pl.BlockSpec((2 * _D, _AUGW), lambda i: (0, 0)),
            pl.BlockSpec((2 * _D, _H), lambda i: (0, 0)),
            pl.BlockSpec((2 * _D, _H), lambda i: (0, 0)),
            pl.BlockSpec((_H, _H), lambda i: (0, 0)),
            pl.BlockSpec((1, _AUGW), lambda i: (0, 0)),
            pl.BlockSpec((1, _H), lambda i: (0, 0)),
            pl.BlockSpec((1, _H), lambda i: (0, 0)),
            pl.BlockSpec((1, _H), lambda i: (0, 0)),
        ],
        out_specs=[
            pl.BlockSpec((_BLK, _AUGW), lambda i: (i, 0)),
            pl.BlockSpec((_BLK, _H), lambda i: (i, 0)),
            pl.BlockSpec((_BLK, _H), lambda i: (i, 0)),
        ],
        out_shape=[
            jax.ShapeDtypeStruct((_N, _AUGW), jnp.float32),
            jax.ShapeDtypeStruct((_N, _H), jnp.float32),
            jax.ShapeDtypeStruct((_N, _H), jnp.float32),
        ],
    )(nf, ef, wla, wr, w1, w2, bcol, blr, b1r, b2r)


def _k34_body(a0, a1, baseb, mlpb, batchb, gam, bet, wf, bfr,
              out, pre_s, sums_s, sumsq_s, gacc, cacc):
    i = pl.program_id(0)

    @pl.when(i == 0)
    def _():
        sums_s[...] = jnp.zeros_like(sums_s)
        sumsq_s[...] = jnp.zeros_like(sumsq_s)
        gacc[...] = jnp.zeros_like(gacc)
        cacc[...] = jnp.zeros_like(cacc)

    @pl.when(i < _NBLK)
    def _():
        aggb = a0[0] + a1[0]
        deg = jnp.maximum(aggb[:, _H:_H + 1], 1.0)
        pre = aggb[:, :_H] / deg + baseb[...]
        pre_s[pl.ds(i * _BLK, _BLK), :] = pre
        sums_s[...] += jnp.sum(pre, axis=0, keepdims=True)
        sumsq_s[...] += jnp.sum(pre * pre, axis=0, keepdims=True)

    @pl.when(i >= _NBLK)
    def _():
        k = i - _NBLK
        mu = sums_s[...] / _N
        var = sumsq_s[...] / _N - mu * mu
        rstd = lax.rsqrt(var + 1e-5)
        preb = pre_s[pl.ds(k * _BLK, _BLK), :]
        xg = (preb - mu) * rstd * gam[...] + bet[...]
        comb = jnp.maximum(xg, 0.0) + mlpb[...]
        b = batchb[0]
        gi = lax.broadcasted_iota(jnp.int32, (_G, 1), 0)
        oh = (gi == b).astype(jnp.float32)
        gacc[...] += jnp.dot(oh, comb, preferred_element_type=jnp.float32)
        cacc[...] += jnp.sum(oh, axis=1, keepdims=True)

    @pl.when(i == 2 * _NBLK - 1)
    def _():
        gf = gacc[...] / jnp.maximum(cacc[...], 1.0)
        z = jnp.dot(gf, wf[...], preferred_element_type=jnp.float32) + bfr[...]
        out[...] = jax.nn.sigmoid(z)


def _k34(agg2, base, mlp, batch3, gam, bet, wf, bfr):
    ilo = lambda i: jnp.minimum(i, _NBLK - 1)
    ihi = lambda i: jnp.maximum(i - _NBLK, 0)
    return pl.pallas_call(
        _k34_body,
        grid=(2 * _NBLK,),
        in_specs=[
            pl.BlockSpec((1, _BLK, _AUGW), lambda i: (0, ilo(i), 0)),
            pl.BlockSpec((1, _BLK, _AUGW), lambda i: (1, ilo(i), 0)),
            pl.BlockSpec((_BLK, _H), lambda i: (ilo(i), 0)),
            pl.BlockSpec((_BLK, _H), lambda i: (ihi(i), 0)),
            pl.BlockSpec((1, 1, _BLK), lambda i: (ihi(i), 0, 0)),
            pl.BlockSpec((1, _H), lambda i: (0, 0)),
            pl.BlockSpec((1, _H), lambda i: (0, 0)),
            pl.BlockSpec((_H, 1), lambda i: (0, 0)),
            pl.BlockSpec((1, 1), lambda i: (0, 0)),
        ],
        out_specs=pl.BlockSpec((_G, 1), lambda i: (0, 0)),
        out_shape=jax.ShapeDtypeStruct((_G, 1), jnp.float32),
        scratch_shapes=[
            pltpu.VMEM((_N, _H), jnp.float32),
            pltpu.VMEM((1, _H), jnp.float32),
            pltpu.VMEM((1, _H), jnp.float32),
            pltpu.VMEM((_G, _H), jnp.float32),
            pltpu.VMEM((_G, 1), jnp.float32),
        ],
    )(agg2, agg2, base, mlp, batch3, gam, bet, wf, bfr)


def kernel(normal_features, extreme_features, Wl, bl, Wr, gamma, beta,
           W1, b1, W2, b2, Wf, bf, edge_index, batch):
    f32 = jnp.float32
    wla = jnp.concatenate([Wl, jnp.zeros((2 * _D, _AUGW - _H), f32)], axis=1)
    bcol = jnp.zeros((1, _AUGW), f32).at[0, _H].set(1.0)
    blr = bl.reshape(1, _H)
    b1r = b1.reshape(1, _H)
    b2r = b2.reshape(1, _H)
    bfr = bf.reshape(1, 1)
    gam = gamma.reshape(1, _H)
    bet = beta.reshape(1, _H)

    yl_aug, base, mlp = _k1(normal_features, extreme_features,
                            wla, Wr, W1, W2, bcol, blr, b1r, b2r)

    zeros = jnp.zeros((_ROWS_PER_TILE, _AUGW), f32)
    agg2 = _edge_agg(yl_aug, edge_index[0], edge_index[1], zeros)

    batch3 = batch.reshape(_NBLK, 1, _BLK)
    return _k34(agg2, base, mlp, batch3, gam, bet, Wf, bfr)


# speedup vs baseline: 2.6320x; 1.0274x over previous
import functools

import jax
import jax.numpy as jnp
from jax import lax
from jax.experimental import pallas as pl
from jax.experimental.pallas import tpu as pltpu
from jax.experimental.pallas import tpu_sc as plsc

_N = 10000
_E = 320000
_D = 128
_H = 128
_G = 64
_AUGW = 160
_CHUNK = 128
_NCHUNKS = _E // _CHUNK
_NW = 32
_NPAD = 10240
_ROWS_PER_TILE = _NPAD // 16
_BLK = 1000
_NBLK = _N // _BLK


_NCH_W = 78


def _sc_edge_agg_body(yl_hbm, src_hbm, dst_hbm, zeros_hbm, out_hbm,
                      src_a, dst_a, src_b, dst_b,
                      buf_a, buf_b, agg_sh, sem_a, sem_b):
    c = lax.axis_index("c")
    s = lax.axis_index("s")
    wid = s * 2 + c
    pltpu.sync_copy(zeros_hbm,
                    agg_sh.at[pl.ds(s * _ROWS_PER_TILE, _ROWS_PER_TILE)])
    plsc.subcore_barrier()

    ebase = wid * (_NCH_W * _CHUNK)

    def load_idx_abs(base, sref, dref):
        pltpu.sync_copy(src_hbm.at[pl.ds(base, _CHUNK)], sref)
        pltpu.sync_copy(dst_hbm.at[pl.ds(base, _CHUNK)], dref)

    def load_idx(j, sref, dref):
        load_idx_abs(ebase + j * _CHUNK, sref, dref)

    def start_gather(sref, buf, sem):
        pltpu.async_copy(yl_hbm.at[sref], buf, sem)

    def wait_gather(sref, buf, sem):
        pltpu.make_async_copy(yl_hbm.at[sref], buf, sem).wait()

    load_idx(0, src_a, dst_a)
    start_gather(src_a, buf_a, sem_a)

    def body(t, carry):
        j1 = 2 * t + 1
        load_idx(j1, src_b, dst_b)
        start_gather(src_b, buf_b, sem_b)
        wait_gather(src_a, buf_a, sem_a)
        pltpu.sync_copy(buf_a, agg_sh.at[dst_a], add=True)
        load_idx(j1 + 1, src_a, dst_a)
        start_gather(src_a, buf_a, sem_a)
        wait_gather(src_b, buf_b, sem_b)
        pltpu.sync_copy(buf_b, agg_sh.at[dst_b], add=True)
        return carry

    lax.fori_loop(0, _NCH_W // 2 - 1, body, 0)
    load_idx(_NCH_W - 1, src_b, dst_b)
    start_gather(src_b, buf_b, sem_b)
    wait_gather(src_a, buf_a, sem_a)
    pltpu.sync_copy(buf_a, agg_sh.at[dst_a], add=True)
    wait_gather(src_b, buf_b, sem_b)
    pltpu.sync_copy(buf_b, agg_sh.at[dst_b], add=True)

    @pl.when(wid < 4)
    def _():
        load_idx_abs((_NW * _NCH_W + wid) * _CHUNK, src_a, dst_a)
        pltpu.async_copy(yl_hbm.at[src_a], buf_a, sem_a).wait()
        pltpu.sync_copy(buf_a, agg_sh.at[dst_a], add=True)

    plsc.subcore_barrier()
    pltpu.sync_copy(agg_sh.at[pl.ds(s * _ROWS_PER_TILE, _ROWS_PER_TILE)],
                    out_hbm.at[c, pl.ds(s * _ROWS_PER_TILE, _ROWS_PER_TILE)])


def _edge_agg(yl_aug, src1, dst1, zeros):
    call = pl.kernel(
        _sc_edge_agg_body,
        out_type=jax.ShapeDtypeStruct((2, _NPAD, _AUGW), jnp.bfloat16),
        mesh=plsc.VectorSubcoreMesh(core_axis_name="c", subcore_axis_name="s"),
        scratch_types=[
            pltpu.VMEM((_CHUNK,), jnp.int32),
            pltpu.VMEM((_CHUNK,), jnp.int32),
            pltpu.VMEM((_CHUNK,), jnp.int32),
            pltpu.VMEM((_CHUNK,), jnp.int32),
            pltpu.VMEM((_CHUNK, _AUGW), jnp.bfloat16),
            pltpu.VMEM((_CHUNK, _AUGW), jnp.bfloat16),
            pltpu.VMEM_SHARED((_NPAD, _AUGW), jnp.bfloat16),
            pltpu.SemaphoreType.DMA,
            pltpu.SemaphoreType.DMA,
        ],
        compiler_params=pltpu.CompilerParams(use_tc_tiling_on_sc=False),
    )
    return call(yl_aug, src1, dst1, zeros)


def _k1_body(nb, eb, wla, wr, w1, w2, bcol, blr, b1r, b2r,
             yl_out, base_out, mlp_out):
    xb = jnp.concatenate([nb[...], eb[...]], axis=1)
    yl_out[...] = (jnp.dot(xb, wla[...], preferred_element_type=jnp.float32)
                   + bcol[...]).astype(jnp.bfloat16)
    base_out[...] = jnp.dot(xb, wr[...], preferred_element_type=jnp.float32) + blr[...]
    h1 = jnp.maximum(jnp.dot(xb, w1[...], preferred_element_type=jnp.float32) + b1r[...], 0.0)
    mlp_out[...] = jnp.maximum(jnp.dot(h1, w2[...], preferred_element_type=jnp.float32) + b2r[...], 0.0)


def _k1(nf, ef, wla, wr, w1, w2, bcol, blr, b1r, b2r):
    return pl.pallas_call(
        _k1_body,
        grid=(_NBLK,),
        in_specs=[
            pl.BlockSpec((_BLK, _D), lambda i: (i, 0)),
            pl.BlockSpec((_BLK, _D), lambda i: (i, 0)),
            pl.BlockSpec((2 * _D, _AUGW), lambda i: (0, 0)),
            pl.BlockSpec((2 * _D, _H), lambda i: (0, 0)),
            pl.BlockSpec((2 * _D, _H), lambda i: (0, 0)),
            pl.BlockSpec((_H, _H), lambda i: (0, 0)),
            pl.BlockSpec((1, _AUGW), lambda i: (0, 0)),
            pl.BlockSpec((1, _H), lambda i: (0, 0)),
            pl.BlockSpec((1, _H), lambda i: (0, 0)),
            pl.BlockSpec((1, _H), lambda i: (0, 0)),
        ],
        out_specs=[
            pl.BlockSpec((_BLK, _AUGW), lambda i: (i, 0)),
            pl.BlockSpec((_BLK, _H), lambda i: (i, 0)),
            pl.BlockSpec((_BLK, _H), lambda i: (i, 0)),
        ],
        out_shape=[
            jax.ShapeDtypeStruct((_N, _AUGW), jnp.bfloat16),
            jax.ShapeDtypeStruct((_N, _H), jnp.float32),
            jax.ShapeDtypeStruct((_N, _H), jnp.float32),
        ],
    )(nf, ef, wla, wr, w1, w2, bcol, blr, b1r, b2r)


def _k34_body(a0, a1, baseb, mlpb, batchb, gam, bet, wf, bfr,
              out, pre_s, sums_s, sumsq_s, gacc, cacc):
    i = pl.program_id(0)

    @pl.when(i == 0)
    def _():
        sums_s[...] = jnp.zeros_like(sums_s)
        sumsq_s[...] = jnp.zeros_like(sumsq_s)
        gacc[...] = jnp.zeros_like(gacc)
        cacc[...] = jnp.zeros_like(cacc)

    @pl.when(i < _NBLK)
    def _():
        aggb = a0[0].astype(jnp.float32) + a1[0].astype(jnp.float32)
        deg = jnp.maximum(aggb[:, _H:_H + 1], 1.0)
        pre = aggb[:, :_H] / deg + baseb[...]
        pre_s[pl.ds(i * _BLK, _BLK), :] = pre
        sums_s[...] += jnp.sum(pre, axis=0, keepdims=True)
        sumsq_s[...] += jnp.sum(pre * pre, axis=0, keepdims=True)

    @pl.when(i >= _NBLK)
    def _():
        k = i - _NBLK
        mu = sums_s[...] / _N
        var = sumsq_s[...] / _N - mu * mu
        rstd = lax.rsqrt(var + 1e-5)
        preb = pre_s[pl.ds(k * _BLK, _BLK), :]
        xg = (preb - mu) * rstd * gam[...] + bet[...]
        comb = jnp.maximum(xg, 0.0) + mlpb[...]
        b = batchb[0]
        gi = lax.broadcasted_iota(jnp.int32, (_G, 1), 0)
        oh = (gi == b).astype(jnp.float32)
        gacc[...] += jnp.dot(oh, comb, preferred_element_type=jnp.float32)
        cacc[...] += jnp.sum(oh, axis=1, keepdims=True)

    @pl.when(i == 2 * _NBLK - 1)
    def _():
        gf = gacc[...] / jnp.maximum(cacc[...], 1.0)
        z = jnp.dot(gf, wf[...], preferred_element_type=jnp.float32) + bfr[...]
        out[...] = jax.nn.sigmoid(z)


def _k34(agg2, base, mlp, batch3, gam, bet, wf, bfr):
    ilo = lambda i: jnp.minimum(i, _NBLK - 1)
    ihi = lambda i: jnp.maximum(i - _NBLK, 0)
    return pl.pallas_call(
        _k34_body,
        grid=(2 * _NBLK,),
        in_specs=[
            pl.BlockSpec((1, _BLK, _AUGW), lambda i: (0, ilo(i), 0)),
            pl.BlockSpec((1, _BLK, _AUGW), lambda i: (1, ilo(i), 0)),
            pl.BlockSpec((_BLK, _H), lambda i: (ilo(i), 0)),
            pl.BlockSpec((_BLK, _H), lambda i: (ihi(i), 0)),
            pl.BlockSpec((1, 1, _BLK), lambda i: (ihi(i), 0, 0)),
            pl.BlockSpec((1, _H), lambda i: (0, 0)),
            pl.BlockSpec((1, _H), lambda i: (0, 0)),
            pl.BlockSpec((_H, 1), lambda i: (0, 0)),
            pl.BlockSpec((1, 1), lambda i: (0, 0)),
        ],
        out_specs=pl.BlockSpec((_G, 1), lambda i: (0, 0)),
        out_shape=jax.ShapeDtypeStruct((_G, 1), jnp.float32),
        scratch_shapes=[
            pltpu.VMEM((_N, _H), jnp.float32),
            pltpu.VMEM((1, _H), jnp.float32),
            pltpu.VMEM((1, _H), jnp.float32),
            pltpu.VMEM((_G, _H), jnp.float32),
            pltpu.VMEM((_G, 1), jnp.float32),
        ],
    )(agg2, agg2, base, mlp, batch3, gam, bet, wf, bfr)


def kernel(normal_features, extreme_features, Wl, bl, Wr, gamma, beta,
           W1, b1, W2, b2, Wf, bf, edge_index, batch):
    f32 = jnp.float32
    wla = jnp.concatenate([Wl, jnp.zeros((2 * _D, _AUGW - _H), f32)], axis=1)
    bcol = jnp.zeros((1, _AUGW), f32).at[0, _H].set(1.0)
    blr = bl.reshape(1, _H)
    b1r = b1.reshape(1, _H)
    b2r = b2.reshape(1, _H)
    bfr = bf.reshape(1, 1)
    gam = gamma.reshape(1, _H)
    bet = beta.reshape(1, _H)

    yl_aug, base, mlp = _k1(normal_features, extreme_features,
                            wla, Wr, W1, W2, bcol, blr, b1r, b2r)

    zeros = jnp.zeros((_ROWS_PER_TILE, _AUGW), jnp.bfloat16)
    agg2 = _edge_agg(yl_aug, edge_index[0], edge_index[1], zeros)

    batch3 = batch.reshape(_NBLK, 1, _BLK)
    return _k34(agg2, base, mlp, batch3, gam, bet, Wf, bfr)


# speedup vs baseline: 3.1598x; 1.2005x over previous
import functools

import jax
import jax.numpy as jnp
from jax import lax
from jax.experimental import pallas as pl
from jax.experimental.pallas import tpu as pltpu
from jax.experimental.pallas import tpu_sc as plsc

_N = 10000
_E = 320000
_D = 128
_H = 128
_G = 64
_AUGW = 160
_CHUNK = 128
_NCHUNKS = _E // _CHUNK
_NW = 32
_NPAD = 10240
_ROWS_PER_TILE = _NPAD // 16
_BLK = 1000
_NBLK = _N // _BLK


_NCH_W = 78


def _sc_edge_agg_body(yl_hbm, eidx_hbm, zeros_hbm, out_hbm,
                      ia, ib, buf_a, buf_b, agg_sh, sem_a, sem_b, sem_i):
    c = lax.axis_index("c")
    s = lax.axis_index("s")
    wid = s * 2 + c
    pltpu.sync_copy(zeros_hbm,
                    agg_sh.at[pl.ds(s * _ROWS_PER_TILE, _ROWS_PER_TILE)])
    plsc.subcore_barrier()

    cbase = wid * _NCH_W

    def start_idx(j, iref):
        pltpu.async_copy(eidx_hbm.at[j], iref, sem_i)

    def wait_idx(iref):
        pltpu.make_async_copy(eidx_hbm.at[0], iref, sem_i).wait()

    def start_gather(iref, buf, sem):
        pltpu.async_copy(yl_hbm.at[iref.at[0]], buf, sem)

    def wait_gather(iref, buf, sem):
        pltpu.make_async_copy(yl_hbm.at[iref.at[0]], buf, sem).wait()

    def scatter(iref, buf):
        pltpu.sync_copy(buf, agg_sh.at[iref.at[1]], add=True)

    start_idx(cbase, ia)
    wait_idx(ia)
    start_gather(ia, buf_a, sem_a)
    start_idx(cbase + 1, ib)

    def body(t, carry):
        j0 = cbase + 2 * t
        wait_idx(ib)
        start_gather(ib, buf_b, sem_b)
        wait_gather(ia, buf_a, sem_a)
        scatter(ia, buf_a)
        start_idx(j0 + 2, ia)
        wait_gather(ib, buf_b, sem_b)
        scatter(ib, buf_b)
        wait_idx(ia)
        start_gather(ia, buf_a, sem_a)
        start_idx(j0 + 3, ib)
        return carry

    lax.fori_loop(0, _NCH_W // 2 - 1, body, 0)
    wait_idx(ib)
    start_gather(ib, buf_b, sem_b)
    wait_gather(ia, buf_a, sem_a)
    scatter(ia, buf_a)
    wait_gather(ib, buf_b, sem_b)
    scatter(ib, buf_b)

    @pl.when(wid < 4)
    def _():
        start_idx(_NW * _NCH_W + wid, ia)
        wait_idx(ia)
        pltpu.async_copy(yl_hbm.at[ia.at[0]], buf_a, sem_a).wait()
        scatter(ia, buf_a)

    plsc.subcore_barrier()
    pltpu.sync_copy(agg_sh.at[pl.ds(s * _ROWS_PER_TILE, _ROWS_PER_TILE)],
                    out_hbm.at[c, pl.ds(s * _ROWS_PER_TILE, _ROWS_PER_TILE)])


def _edge_agg(yl_aug, eidx3, zeros):
    call = pl.kernel(
        _sc_edge_agg_body,
        out_type=jax.ShapeDtypeStruct((2, _NPAD, _AUGW), jnp.bfloat16),
        mesh=plsc.VectorSubcoreMesh(core_axis_name="c", subcore_axis_name="s"),
        scratch_types=[
            pltpu.VMEM((2, _CHUNK), jnp.int32),
            pltpu.VMEM((2, _CHUNK), jnp.int32),
            pltpu.VMEM((_CHUNK, _AUGW), jnp.bfloat16),
            pltpu.VMEM((_CHUNK, _AUGW), jnp.bfloat16),
            pltpu.VMEM_SHARED((_NPAD, _AUGW), jnp.bfloat16),
            pltpu.SemaphoreType.DMA,
            pltpu.SemaphoreType.DMA,
            pltpu.SemaphoreType.DMA,
        ],
        compiler_params=pltpu.CompilerParams(use_tc_tiling_on_sc=False),
    )
    return call(yl_aug, eidx3, zeros)


def _k1_body(nb, eb, wla, wr, w1, w2, bcol, blr, b1r, b2r,
             yl_out, base_out, mlp_out):
    xb = jnp.concatenate([nb[...], eb[...]], axis=1)
    yl_out[...] = (jnp.dot(xb, wla[...], preferred_element_type=jnp.float32)
                   + bcol[...]).astype(jnp.bfloat16)
    base_out[...] = jnp.dot(xb, wr[...], preferred_element_type=jnp.float32) + blr[...]
    h1 = jnp.maximum(jnp.dot(xb, w1[...], preferred_element_type=jnp.float32) + b1r[...], 0.0)
    mlp_out[...] = jnp.maximum(jnp.dot(h1, w2[...], preferred_element_type=jnp.float32) + b2r[...], 0.0)


def _k1(nf, ef, wla, wr, w1, w2, bcol, blr, b1r, b2r):
    return pl.pallas_call(
        _k1_body,
        grid=(_NBLK,),
        in_specs=[
            pl.BlockSpec((_BLK, _D), lambda i: (i, 0)),
            pl.BlockSpec((_BLK, _D), lambda i: (i, 0)),
            pl.BlockSpec((2 * _D, _AUGW), lambda i: (0, 0)),
            pl.BlockSpec((2 * _D, _H), lambda i: (0, 0)),
            pl.BlockSpec((2 * _D, _H), lambda i: (0, 0)),
            pl.BlockSpec((_H, _H), lambda i: (0, 0)),
            pl.BlockSpec((1, _AUGW), lambda i: (0, 0)),
            pl.BlockSpec((1, _H), lambda i: (0, 0)),
            pl.BlockSpec((1, _H), lambda i: (0, 0)),
            pl.BlockSpec((1, _H), lambda i: (0, 0)),
        ],
        out_specs=[
            pl.BlockSpec((_BLK, _AUGW), lambda i: (i, 0)),
            pl.BlockSpec((_BLK, _H), lambda i: (i, 0)),
            pl.BlockSpec((_BLK, _H), lambda i: (i, 0)),
        ],
        out_shape=[
            jax.ShapeDtypeStruct((_N, _AUGW), jnp.bfloat16),
            jax.ShapeDtypeStruct((_N, _H), jnp.float32),
            jax.ShapeDtypeStruct((_N, _H), jnp.float32),
        ],
    )(nf, ef, wla, wr, w1, w2, bcol, blr, b1r, b2r)


def _k34_body(a0, a1, baseb, mlpb, batchb, gam, bet, wf, bfr,
              out, pre_s, sums_s, sumsq_s, gacc, cacc):
    i = pl.program_id(0)

    @pl.when(i == 0)
    def _():
        sums_s[...] = jnp.zeros_like(sums_s)
        sumsq_s[...] = jnp.zeros_like(sumsq_s)
        gacc[...] = jnp.zeros_like(gacc)
        cacc[...] = jnp.zeros_like(cacc)

    @pl.when(i < _NBLK)
    def _():
        aggb = a0[0].astype(jnp.float32) + a1[0].astype(jnp.float32)
        deg = jnp.maximum(aggb[:, _H:_H + 1], 1.0)
        pre = aggb[:, :_H] / deg + baseb[...]
        pre_s[pl.ds(i * _BLK, _BLK), :] = pre
        sums_s[...] += jnp.sum(pre, axis=0, keepdims=True)
        sumsq_s[...] += jnp.sum(pre * pre, axis=0, keepdims=True)

    @pl.when(i >= _NBLK)
    def _():
        k = i - _NBLK
        mu = sums_s[...] / _N
        var = sumsq_s[...] / _N - mu * mu
        rstd = lax.rsqrt(var + 1e-5)
        preb = pre_s[pl.ds(k * _BLK, _BLK), :]
        xg = (preb - mu) * rstd * gam[...] + bet[...]
        comb = jnp.maximum(xg, 0.0) + mlpb[...]
        b = batchb[0]
        gi = lax.broadcasted_iota(jnp.int32, (_G, 1), 0)
        oh = (gi == b).astype(jnp.float32)
        gacc[...] += jnp.dot(oh, comb, preferred_element_type=jnp.float32)
        cacc[...] += jnp.sum(oh, axis=1, keepdims=True)

    @pl.when(i == 2 * _NBLK - 1)
    def _():
        gf = gacc[...] / jnp.maximum(cacc[...], 1.0)
        z = jnp.dot(gf, wf[...], preferred_element_type=jnp.float32) + bfr[...]
        out[...] = jax.nn.sigmoid(z)


def _k34(agg2, base, mlp, batch3, gam, bet, wf, bfr):
    ilo = lambda i: jnp.minimum(i, _NBLK - 1)
    ihi = lambda i: jnp.maximum(i - _NBLK, 0)
    return pl.pallas_call(
        _k34_body,
        grid=(2 * _NBLK,),
        in_specs=[
            pl.BlockSpec((1, _BLK, _AUGW), lambda i: (0, ilo(i), 0)),
            pl.BlockSpec((1, _BLK, _AUGW), lambda i: (1, ilo(i), 0)),
            pl.BlockSpec((_BLK, _H), lambda i: (ilo(i), 0)),
            pl.BlockSpec((_BLK, _H), lambda i: (ihi(i), 0)),
            pl.BlockSpec((1, 1, _BLK), lambda i: (ihi(i), 0, 0)),
            pl.BlockSpec((1, _H), lambda i: (0, 0)),
            pl.BlockSpec((1, _H), lambda i: (0, 0)),
            pl.BlockSpec((_H, 1), lambda i: (0, 0)),
            pl.BlockSpec((1, 1), lambda i: (0, 0)),
        ],
        out_specs=pl.BlockSpec((_G, 1), lambda i: (0, 0)),
        out_shape=jax.ShapeDtypeStruct((_G, 1), jnp.float32),
        scratch_shapes=[
            pltpu.VMEM((_N, _H), jnp.float32),
            pltpu.VMEM((1, _H), jnp.float32),
            pltpu.VMEM((1, _H), jnp.float32),
            pltpu.VMEM((_G, _H), jnp.float32),
            pltpu.VMEM((_G, 1), jnp.float32),
        ],
    )(agg2, agg2, base, mlp, batch3, gam, bet, wf, bfr)


def kernel(normal_features, extreme_features, Wl, bl, Wr, gamma, beta,
           W1, b1, W2, b2, Wf, bf, edge_index, batch):
    f32 = jnp.float32
    wla = jnp.concatenate([Wl, jnp.zeros((2 * _D, _AUGW - _H), f32)], axis=1)
    bcol = jnp.zeros((1, _AUGW), f32).at[0, _H].set(1.0)
    blr = bl.reshape(1, _H)
    b1r = b1.reshape(1, _H)
    b2r = b2.reshape(1, _H)
    bfr = bf.reshape(1, 1)
    gam = gamma.reshape(1, _H)
    bet = beta.reshape(1, _H)

    yl_aug, base, mlp = _k1(normal_features, extreme_features,
                            wla, Wr, W1, W2, bcol, blr, b1r, b2r)

    zeros = jnp.zeros((_ROWS_PER_TILE, _AUGW), jnp.bfloat16)
    eidx3 = edge_index.reshape(2, _NCHUNKS, _CHUNK).transpose(1, 0, 2)
    agg2 = _edge_agg(yl_aug, eidx3, zeros)

    batch3 = batch.reshape(_NBLK, 1, _BLK)
    return _k34(agg2, base, mlp, batch3, gam, bet, Wf, bfr)


# speedup vs baseline: 3.4891x; 1.1042x over previous
import functools

import jax
import jax.numpy as jnp
from jax import lax
from jax.experimental import pallas as pl
from jax.experimental.pallas import tpu as pltpu
from jax.experimental.pallas import tpu_sc as plsc

_N = 10000
_E = 320000
_D = 128
_H = 128
_G = 64
_AUGW = 160
_CHUNK = 128
_NCHUNKS = _E // _CHUNK
_NW = 32
_NPAD = 10240
_ROWS_PER_TILE = _NPAD // 16
_BLK = 1000
_NBLK = _N // _BLK


_NCH_W = 78


def _sc_edge_agg_body(yl_hbm, eidx_hbm, zeros_hbm, out_hbm,
                      i0, i1, i2, i3, b0, b1, b2, b3, agg_sh,
                      g0, g1, g2, g3, s0, s1, s2, s3):
    c = lax.axis_index("c")
    s = lax.axis_index("s")
    wid = s * 2 + c
    pltpu.sync_copy(zeros_hbm,
                    agg_sh.at[pl.ds(s * _ROWS_PER_TILE, _ROWS_PER_TILE)])
    plsc.subcore_barrier()

    idxs = [i0, i1, i2, i3]
    bufs = [b0, b1, b2, b3]
    gsem = [g0, g1, g2, g3]
    isem = [s0, s1, s2, s3]
    cbase = wid * _NCH_W

    def start_idx(j, k):
        pltpu.async_copy(eidx_hbm.at[j], idxs[k], isem[k])

    def wait_idx(k):
        pltpu.make_async_copy(eidx_hbm.at[0], idxs[k], isem[k]).wait()

    def start_gather(k):
        pltpu.async_copy(yl_hbm.at[idxs[k].at[0]], bufs[k], gsem[k])

    def wait_gather(k):
        pltpu.make_async_copy(yl_hbm.at[idxs[k].at[0]], bufs[k],
                              gsem[k]).wait()

    def scatter(k):
        pltpu.sync_copy(bufs[k], agg_sh.at[idxs[k].at[1]], add=True)

    for k in range(4):
        start_idx(cbase + k, k)
    for k in range(3):
        wait_idx(k)
        start_gather(k)

    def body(t, carry):
        j = cbase + 4 * t
        wait_idx(3)
        start_gather(3)
        for k in range(3):
            wait_gather(k)
            scatter(k)
            start_idx(j + 4 + k, k)
        wait_idx(0)
        start_gather(0)
        wait_idx(1)
        start_gather(1)
        wait_gather(3)
        scatter(3)
        start_idx(j + 7, 3)
        wait_idx(2)
        start_gather(2)
        return carry

    lax.fori_loop(0, (_NCH_W - 2) // 4, body, 0)
    wait_gather(0)
    scatter(0)
    wait_gather(1)
    scatter(1)
    wait_gather(2)
    wait_idx(3)

    @pl.when(wid < 4)
    def _():
        start_idx(_NW * _NCH_W + wid, 0)
        wait_idx(0)
        start_gather(0)
        wait_gather(0)
        scatter(0)

    plsc.subcore_barrier()
    pltpu.sync_copy(agg_sh.at[pl.ds(s * _ROWS_PER_TILE, _ROWS_PER_TILE)],
                    out_hbm.at[c, pl.ds(s * _ROWS_PER_TILE, _ROWS_PER_TILE)])


def _edge_agg(yl_aug, eidx3, zeros):
    call = pl.kernel(
        _sc_edge_agg_body,
        out_type=jax.ShapeDtypeStruct((2, _NPAD, _AUGW), jnp.bfloat16),
        mesh=plsc.VectorSubcoreMesh(core_axis_name="c", subcore_axis_name="s"),
        scratch_types=(
            [pltpu.VMEM((2, _CHUNK), jnp.int32)] * 4
            + [pltpu.VMEM((_CHUNK, _AUGW), jnp.bfloat16)] * 4
            + [pltpu.VMEM_SHARED((_NPAD, _AUGW), jnp.bfloat16)]
            + [pltpu.SemaphoreType.DMA] * 8
        ),
        compiler_params=pltpu.CompilerParams(use_tc_tiling_on_sc=False),
    )
    return call(yl_aug, eidx3, zeros)


def _k1_body(nb, eb, wla, wr, w1, w2, bcol, blr, b1r, b2r,
             yl_out, base_out, mlp_out):
    xb = jnp.concatenate([nb[...], eb[...]], axis=1)
    yl_out[...] = (jnp.dot(xb, wla[...], preferred_element_type=jnp.float32)
                   + bcol[...]).astype(jnp.bfloat16)
    base_out[...] = jnp.dot(xb, wr[...], preferred_element_type=jnp.float32) + blr[...]
    h1 = jnp.maximum(jnp.dot(xb, w1[...], preferred_element_type=jnp.float32) + b1r[...], 0.0)
    mlp_out[...] = jnp.maximum(jnp.dot(h1, w2[...], preferred_element_type=jnp.float32) + b2r[...], 0.0)


def _k1(nf, ef, wla, wr, w1, w2, bcol, blr, b1r, b2r):
    return pl.pallas_call(
        _k1_body,
        grid=(_NBLK,),
        in_specs=[
            pl.BlockSpec((_BLK, _D), lambda i: (i, 0)),
            pl.BlockSpec((_BLK, _D), lambda i: (i, 0)),
            pl.BlockSpec((2 * _D, _AUGW), lambda i: (0, 0)),
            pl.BlockSpec((2 * _D, _H), lambda i: (0, 0)),
            pl.BlockSpec((2 * _D, _H), lambda i: (0, 0)),
            pl.BlockSpec((_H, _H), lambda i: (0, 0)),
            pl.BlockSpec((1, _AUGW), lambda i: (0, 0)),
            pl.BlockSpec((1, _H), lambda i: (0, 0)),
            pl.BlockSpec((1, _H), lambda i: (0, 0)),
            pl.BlockSpec((1, _H), lambda i: (0, 0)),
        ],
        out_specs=[
            pl.BlockSpec((_BLK, _AUGW), lambda i: (i, 0)),
            pl.BlockSpec((_BLK, _H), lambda i: (i, 0)),
            pl.BlockSpec((_BLK, _H), lambda i: (i, 0)),
        ],
        out_shape=[
            jax.ShapeDtypeStruct((_N, _AUGW), jnp.bfloat16),
            jax.ShapeDtypeStruct((_N, _H), jnp.float32),
            jax.ShapeDtypeStruct((_N, _H), jnp.float32),
        ],
    )(nf, ef, wla, wr, w1, w2, bcol, blr, b1r, b2r)


def _k34_body(a0, a1, baseb, mlpb, batchb, gam, bet, wf, bfr,
              out, pre_s, sums_s, sumsq_s, gacc, cacc):
    i = pl.program_id(0)

    @pl.when(i == 0)
    def _():
        sums_s[...] = jnp.zeros_like(sums_s)
        sumsq_s[...] = jnp.zeros_like(sumsq_s)
        gacc[...] = jnp.zeros_like(gacc)
        cacc[...] = jnp.zeros_like(cacc)

    @pl.when(i < _NBLK)
    def _():
        aggb = a0[0].astype(jnp.float32) + a1[0].astype(jnp.float32)
        deg = jnp.maximum(aggb[:, _H:_H + 1], 1.0)
        pre = aggb[:, :_H] / deg + baseb[...]
        pre_s[pl.ds(i * _BLK, _BLK), :] = pre
        sums_s[...] += jnp.sum(pre, axis=0, keepdims=True)
        sumsq_s[...] += jnp.sum(pre * pre, axis=0, keepdims=True)

    @pl.when(i >= _NBLK)
    def _():
        k = i - _NBLK
        mu = sums_s[...] / _N
        var = sumsq_s[...] / _N - mu * mu
        rstd = lax.rsqrt(var + 1e-5)
        preb = pre_s[pl.ds(k * _BLK, _BLK), :]
        xg = (preb - mu) * rstd * gam[...] + bet[...]
        comb = jnp.maximum(xg, 0.0) + mlpb[...]
        b = batchb[0]
        gi = lax.broadcasted_iota(jnp.int32, (_G, 1), 0)
        oh = (gi == b).astype(jnp.float32)
        gacc[...] += jnp.dot(oh, comb, preferred_element_type=jnp.float32)
        cacc[...] += jnp.sum(oh, axis=1, keepdims=True)

    @pl.when(i == 2 * _NBLK - 1)
    def _():
        gf = gacc[...] / jnp.maximum(cacc[...], 1.0)
        z = jnp.dot(gf, wf[...], preferred_element_type=jnp.float32) + bfr[...]
        out[...] = jax.nn.sigmoid(z)


def _k34(agg2, base, mlp, batch3, gam, bet, wf, bfr):
    ilo = lambda i: jnp.minimum(i, _NBLK - 1)
    ihi = lambda i: jnp.maximum(i - _NBLK, 0)
    return pl.pallas_call(
        _k34_body,
        grid=(2 * _NBLK,),
        in_specs=[
            pl.BlockSpec((1, _BLK, _AUGW), lambda i: (0, ilo(i), 0)),
            pl.BlockSpec((1, _BLK, _AUGW), lambda i: (1, ilo(i), 0)),
            pl.BlockSpec((_BLK, _H), lambda i: (ilo(i), 0)),
            pl.BlockSpec((_BLK, _H), lambda i: (ihi(i), 0)),
            pl.BlockSpec((1, 1, _BLK), lambda i: (ihi(i), 0, 0)),
            pl.BlockSpec((1, _H), lambda i: (0, 0)),
            pl.BlockSpec((1, _H), lambda i: (0, 0)),
            pl.BlockSpec((_H, 1), lambda i: (0, 0)),
            pl.BlockSpec((1, 1), lambda i: (0, 0)),
        ],
        out_specs=pl.BlockSpec((_G, 1), lambda i: (0, 0)),
        out_shape=jax.ShapeDtypeStruct((_G, 1), jnp.float32),
        scratch_shapes=[
            pltpu.VMEM((_N, _H), jnp.float32),
            pltpu.VMEM((1, _H), jnp.float32),
            pltpu.VMEM((1, _H), jnp.float32),
            pltpu.VMEM((_G, _H), jnp.float32),
            pltpu.VMEM((_G, 1), jnp.float32),
        ],
    )(agg2, agg2, base, mlp, batch3, gam, bet, wf, bfr)


def kernel(normal_features, extreme_features, Wl, bl, Wr, gamma, beta,
           W1, b1, W2, b2, Wf, bf, edge_index, batch):
    f32 = jnp.float32
    wla = jnp.concatenate([Wl, jnp.zeros((2 * _D, _AUGW - _H), f32)], axis=1)
    bcol = jnp.zeros((1, _AUGW), f32).at[0, _H].set(1.0)
    blr = bl.reshape(1, _H)
    b1r = b1.reshape(1, _H)
    b2r = b2.reshape(1, _H)
    bfr = bf.reshape(1, 1)
    gam = gamma.reshape(1, _H)
    bet = beta.reshape(1, _H)

    yl_aug, base, mlp = _k1(normal_features, extreme_features,
                            wla, Wr, W1, W2, bcol, blr, b1r, b2r)

    zeros = jnp.zeros((_ROWS_PER_TILE, _AUGW), jnp.bfloat16)
    eidx3 = edge_index.reshape(2, _NCHUNKS, _CHUNK).transpose(1, 0, 2)
    agg2 = _edge_agg(yl_aug, eidx3, zeros)

    batch3 = batch.reshape(_NBLK, 1, _BLK)
    return _k34(agg2, base, mlp, batch3, gam, bet, Wf, bfr)
